# Initial kernel scaffold; baseline (speedup 1.0000x reference)
import jax
import jax.numpy as jnp
from jax.experimental import pallas as pl


def kernel(features, img_info, gt_bboxes, W1, b1, Wc, bc, Wr, br):
    raise NotImplementedError("write your pallas kernel here")


# speedup vs baseline: 5.9570x; 5.9570x over previous
import numpy as np
import jax
import jax.numpy as jnp
from jax.experimental import pallas as pl
from jax.experimental.pallas import tpu as pltpu

_RATIOS = np.array([0.5, 1.0, 2.0], np.float32)
_STRIDE = 8
_SIZE = 64
_K_PRE = 6000
_N_OUT = 300
_TH = 0.7
_H = 64
_W = 64
_A = 3
_C_IN = 256
_C_MID = 512
_N = _H * _W * _A
_ROWS = _N // 128
_MBLK = 1024
_PAD = 72


def _trunk_body(fl_ref, fc_ref, fr_ref, w1_ref, b1_ref, wh_ref, bh_ref, out_ref):
    mb = pl.program_id(1)
    base = pl.multiple_of(_PAD + mb * _MBLK, 8)
    variants = (fl_ref, fc_ref, fr_ref)
    acc = jnp.zeros((_MBLK, _C_MID), jnp.float32)
    for dy in range(3):
        for dx in range(3):
            src = variants[dx]
            start = pl.multiple_of(base + (dy - 1) * _W, 8)
            a = src[0, pl.ds(start, _MBLK), :]
            acc = acc + jnp.dot(a, w1_ref[dy * 3 + dx],
                                preferred_element_type=jnp.float32)
    x = jnp.maximum(acc + b1_ref[0:1, :], 0.0)
    heads = jnp.dot(x, wh_ref[...], preferred_element_type=jnp.float32)
    out_ref[0] = heads + bh_ref[0:1, :]


def _nms_body(info_ref, cls_ref, rcx_ref, rcy_ref, rw_ref, rh_ref,
              aw_ref, ah_ref, acx_ref, acy_ref, orig_ref, out_ref):
    ih = info_ref[0, 0, 0]
    iw = info_ref[0, 0, 1]
    prob = jax.nn.sigmoid(cls_ref[0])
    aw = aw_ref[...]
    ah = ah_ref[...]
    pcx = rcx_ref[0] * aw + acx_ref[...]
    pcy = rcy_ref[0] * ah + acy_ref[...]
    pw = jnp.exp(jnp.clip(rw_ref[0], -10.0, 10.0)) * aw
    ph = jnp.exp(jnp.clip(rh_ref[0], -10.0, 10.0)) * ah
    x1 = jnp.clip(pcx - 0.5 * pw, 0.0, iw)
    y1 = jnp.clip(pcy - 0.5 * ph, 0.0, ih)
    x2 = jnp.clip(pcx + 0.5 * pw, 0.0, iw)
    y2 = jnp.clip(pcy + 0.5 * ph, 0.0, ih)
    area = jnp.maximum(x2 - x1, 0.0) * jnp.maximum(y2 - y1, 0.0)
    orig = orig_ref[...]
    bits = jax.lax.bitcast_convert_type(prob, jnp.int32)

    def bs_body(_, c):
        lo, hi = c
        mid = (lo + hi + jnp.int32(1)) >> 1
        cnt = jnp.sum((bits >= mid).astype(jnp.int32))
        big = cnt >= _K_PRE
        return (jnp.where(big, mid, lo), jnp.where(big, hi, mid - 1))

    t, _ = jax.lax.fori_loop(0, 31, bs_body,
                             (jnp.int32(0), jnp.int32(0x3F800000)))
    n_gt = jnp.sum((bits > t).astype(jnp.int32))
    quota = jnp.int32(_K_PRE) - n_gt

    def bs2_body(_, c):
        lo, hi = c
        mid = (lo + hi) >> 1
        cnt = jnp.sum(((bits == t) & (orig <= mid)).astype(jnp.int32))
        ok = cnt >= quota
        return (jnp.where(ok, lo, mid + 1), jnp.where(ok, mid, hi))

    istar, _ = jax.lax.fori_loop(0, 14, bs2_body,
                                 (jnp.int32(-1), jnp.int32(_N - 1)))
    eligible = (bits > t) | ((bits == t) & (orig <= istar))

    lane = jax.lax.broadcasted_iota(jnp.int32, (1, 128), 1)

    def nms_body(i, alive_i):
        alive = alive_i > 0
        masked = jnp.where(alive, prob, -1.0)
        m = jnp.max(masked)
        valid = m >= 0.0
        sel = alive & (prob == m)
        idxsel = jnp.min(jnp.where(sel, orig, jnp.int32(0x40000000)))
        onehot = sel & (orig == idxsel)
        bx1 = jnp.sum(jnp.where(onehot, x1, 0.0))
        by1 = jnp.sum(jnp.where(onehot, y1, 0.0))
        bx2 = jnp.sum(jnp.where(onehot, x2, 0.0))
        by2 = jnp.sum(jnp.where(onehot, y2, 0.0))
        ba = jnp.maximum(bx2 - bx1, 0.0) * jnp.maximum(by2 - by1, 0.0)
        xx1 = jnp.maximum(bx1, x1)
        yy1 = jnp.maximum(by1, y1)
        xx2 = jnp.minimum(bx2, x2)
        yy2 = jnp.minimum(by2, y2)
        inter = jnp.maximum(xx2 - xx1, 0.0) * jnp.maximum(yy2 - yy1, 0.0)
        iou = inter / jnp.maximum(ba + area - inter, 1e-8)
        newalive = alive & jnp.logical_not(iou > _TH) & jnp.logical_not(onehot)
        alive_i = jnp.where(valid, newalive.astype(jnp.int32), alive_i)
        v0 = jnp.where(valid, bx1, 0.0)
        v1 = jnp.where(valid, by1, 0.0)
        v2 = jnp.where(valid, bx2, 1.0)
        v3 = jnp.where(valid, by2, 1.0)
        row = jnp.where(lane == 0, v0,
                        jnp.where(lane == 1, v1,
                                  jnp.where(lane == 2, v2,
                                            jnp.where(lane == 3, v3, 0.0))))
        out_ref[0, pl.ds(i, 1), :] = row
        return alive_i

    jax.lax.fori_loop(0, _N_OUT, nms_body, eligible.astype(jnp.int32))


def _anchor_consts():
    ratios = _RATIOS.astype(np.float32)
    ws = (np.float32(_SIZE) / np.sqrt(ratios)).astype(np.float32)
    hs = (np.float32(_SIZE) * np.sqrt(ratios)).astype(np.float32)
    sx = (np.arange(_W, dtype=np.float32) + np.float32(0.5)) * np.float32(_STRIDE)
    sy = (np.arange(_H, dtype=np.float32) + np.float32(0.5)) * np.float32(_STRIDE)
    cy, cx = np.meshgrid(sy, sx, indexing='ij')
    cx = cx[:, :, None].astype(np.float32)
    cy = cy[:, :, None].astype(np.float32)
    x1 = cx - ws[None, None, :] * np.float32(0.5)
    y1 = cy - hs[None, None, :] * np.float32(0.5)
    x2 = cx + ws[None, None, :] * np.float32(0.5)
    y2 = cy + hs[None, None, :] * np.float32(0.5)
    anch = np.stack([x1, y1, x2, y2], axis=-1).astype(np.float32).reshape(-1, 4)
    mine = anch.reshape(_H * _W, _A, 4).transpose(1, 0, 2).reshape(_N, 4)
    w = mine[:, 2] - mine[:, 0]
    h = mine[:, 3] - mine[:, 1]
    acx = mine[:, 0] + np.float32(0.5) * w
    acy = mine[:, 1] + np.float32(0.5) * h
    k = np.arange(_N, dtype=np.int32)
    orig = (k % (_H * _W)) * _A + k // (_H * _W)
    r = lambda v: v.reshape(_ROWS, 128)
    return (r(w.astype(np.float32)), r(h.astype(np.float32)),
            r(acx.astype(np.float32)), r(acy.astype(np.float32)),
            r(orig.astype(np.int32)))


_AW, _AH, _ACX, _ACY, _ORIG = _anchor_consts()


def kernel(features, img_info, gt_bboxes, W1, b1, Wc, bc, Wr, br):
    B = features.shape[0]
    feat = jnp.transpose(features, (0, 2, 3, 1)).reshape(B, _H * _W, _C_IN)
    col = jnp.arange(_H * _W) % _W
    fl = jnp.where((col == _W - 1)[None, :, None], 0.0, feat)
    fr = jnp.where((col == 0)[None, :, None], 0.0, feat)
    fl = jnp.pad(fl, ((0, 0), (_PAD + 1, _PAD - 1), (0, 0)))
    fc = jnp.pad(feat, ((0, 0), (_PAD, _PAD), (0, 0)))
    fr = jnp.pad(fr, ((0, 0), (_PAD - 1, _PAD + 1), (0, 0)))
    w1s = jnp.stack([W1[:, :, dy, dx].T for dy in range(3) for dx in range(3)])
    wh = jnp.concatenate([Wc.reshape(_A, _C_MID), Wr.reshape(4 * _A, _C_MID)], 0).T
    wh = jnp.pad(wh, ((0, 0), (0, 128 - 5 * _A)))
    bh = jnp.pad(jnp.concatenate([bc, br]), (0, 128 - 5 * _A)).reshape(1, 128)
    b1r = b1.reshape(1, _C_MID)

    npad = _H * _W + 2 * _PAD
    heads = pl.pallas_call(
        _trunk_body,
        grid=(B, _H * _W // _MBLK),
        in_specs=[
            pl.BlockSpec((1, npad, _C_IN), lambda b, m: (b, 0, 0)),
            pl.BlockSpec((1, npad, _C_IN), lambda b, m: (b, 0, 0)),
            pl.BlockSpec((1, npad, _C_IN), lambda b, m: (b, 0, 0)),
            pl.BlockSpec((9, _C_IN, _C_MID), lambda b, m: (0, 0, 0)),
            pl.BlockSpec((1, _C_MID), lambda b, m: (0, 0)),
            pl.BlockSpec((_C_MID, 128), lambda b, m: (0, 0)),
            pl.BlockSpec((1, 128), lambda b, m: (0, 0)),
        ],
        out_specs=pl.BlockSpec((1, _MBLK, 128), lambda b, m: (b, m, 0)),
        out_shape=jax.ShapeDtypeStruct((B, _H * _W, 128), jnp.float32),
    )(fl, fc, fr, w1s, b1r, wh, bh)

    tr = lambda s: heads[:, :, s].transpose(0, 2, 1).reshape(B, _ROWS, 128)
    cls_my = tr(slice(0, 3))
    rcx = tr(slice(3, 15, 4))
    rcy = tr(slice(4, 15, 4))
    rw = tr(slice(5, 15, 4))
    rh = tr(slice(6, 15, 4))

    cspec = pl.BlockSpec((_ROWS, 128), lambda b: (0, 0))
    bspec = pl.BlockSpec((1, _ROWS, 128), lambda b: (b, 0, 0))
    out = pl.pallas_call(
        _nms_body,
        grid=(B,),
        in_specs=[
            pl.BlockSpec((1, 1, 3), lambda b: (b, 0, 0), memory_space=pltpu.SMEM),
            bspec, bspec, bspec, bspec, bspec,
            cspec, cspec, cspec, cspec, cspec,
        ],
        out_specs=pl.BlockSpec((1, 304, 128), lambda b: (b, 0, 0)),
        out_shape=jax.ShapeDtypeStruct((B, 304, 128), jnp.float32),
    )(img_info.reshape(B, 1, 3), cls_my, rcx, rcy, rw, rh,
      jnp.asarray(_AW), jnp.asarray(_AH), jnp.asarray(_ACX), jnp.asarray(_ACY),
      jnp.asarray(_ORIG))
    return out[:, :_N_OUT, :4]


# speedup vs baseline: 6.7714x; 1.1367x over previous
import numpy as np
import jax
import jax.numpy as jnp
from jax.experimental import pallas as pl
from jax.experimental.pallas import tpu as pltpu

_RATIOS = np.array([0.5, 1.0, 2.0], np.float32)
_STRIDE = 8
_SIZE = 64
_K_PRE = 6000
_N_OUT = 300
_TH = 0.7
_H = 64
_W = 64
_A = 3
_C_IN = 256
_C_MID = 512
_N = _H * _W * _A
_ROWS = _N // 128
_MBLK = 1024
_PAD = 72


def _trunk_body(fl_ref, fc_ref, fr_ref, w1_ref, b1_ref, wh_ref, bh_ref, out_ref):
    mb = pl.program_id(1)
    base = pl.multiple_of(_PAD + mb * _MBLK, 8)
    variants = (fl_ref, fc_ref, fr_ref)
    acc = jnp.zeros((_MBLK, _C_MID), jnp.float32)
    for dy in range(3):
        for dx in range(3):
            src = variants[dx]
            start = pl.multiple_of(base + (dy - 1) * _W, 8)
            a = src[0, pl.ds(start, _MBLK), :]
            acc = acc + jnp.dot(a, w1_ref[dy * 3 + dx],
                                preferred_element_type=jnp.float32)
    x = jnp.maximum(acc + b1_ref[0:1, :], 0.0)
    heads = jnp.dot(x, wh_ref[...], preferred_element_type=jnp.float32)
    out_ref[0] = heads + bh_ref[0:1, :]


def _nms_body(info_ref, cls_ref, rcx_ref, rcy_ref, rw_ref, rh_ref,
              aw_ref, ah_ref, acx_ref, acy_ref, orig_ref, out_ref):
    nb = cls_ref.shape[0]
    aw = aw_ref[...]
    ah = ah_ref[...]
    acx = acx_ref[...]
    acy = acy_ref[...]
    orig = orig_ref[...]
    prob, x1, y1, x2, y2, area, bits = [], [], [], [], [], [], []
    for b in range(nb):
        ih = info_ref[b, 0, 0]
        iw = info_ref[b, 0, 1]
        p = jax.nn.sigmoid(cls_ref[b])
        pcx = rcx_ref[b] * aw + acx
        pcy = rcy_ref[b] * ah + acy
        pw = jnp.exp(jnp.clip(rw_ref[b], -10.0, 10.0)) * aw
        ph = jnp.exp(jnp.clip(rh_ref[b], -10.0, 10.0)) * ah
        bx1 = jnp.clip(pcx - 0.5 * pw, 0.0, iw)
        by1 = jnp.clip(pcy - 0.5 * ph, 0.0, ih)
        bx2 = jnp.clip(pcx + 0.5 * pw, 0.0, iw)
        by2 = jnp.clip(pcy + 0.5 * ph, 0.0, ih)
        prob.append(p)
        x1.append(bx1)
        y1.append(by1)
        x2.append(bx2)
        y2.append(by2)
        area.append(jnp.maximum(bx2 - bx1, 0.0) * jnp.maximum(by2 - by1, 0.0))
        bits.append(jax.lax.bitcast_convert_type(p, jnp.int32))

    def bs_body(_, c):
        out = []
        for b in range(nb):
            lo, hi = c[2 * b], c[2 * b + 1]
            mid = (lo + hi + jnp.int32(1)) >> 1
            cnt = jnp.sum((bits[b] >= mid).astype(jnp.int32))
            big = cnt >= _K_PRE
            out += [jnp.where(big, mid, lo), jnp.where(big, hi, mid - 1)]
        return tuple(out)

    c0 = (jnp.int32(0), jnp.int32(0x3F800000)) * nb
    cr = jax.lax.fori_loop(0, 31, bs_body, c0)
    t = [cr[2 * b] for b in range(nb)]
    quota = [jnp.int32(_K_PRE) - jnp.sum((bits[b] > t[b]).astype(jnp.int32))
             for b in range(nb)]

    def bs2_body(_, c):
        out = []
        for b in range(nb):
            lo, hi = c[2 * b], c[2 * b + 1]
            mid = (lo + hi) >> 1
            cnt = jnp.sum(((bits[b] == t[b]) & (orig <= mid)).astype(jnp.int32))
            ok = cnt >= quota[b]
            out += [jnp.where(ok, lo, mid + 1), jnp.where(ok, mid, hi)]
        return tuple(out)

    c0 = (jnp.int32(-1), jnp.int32(_N - 1)) * nb
    cr = jax.lax.fori_loop(0, 14, bs2_body, c0)
    score0 = []
    for b in range(nb):
        istar = cr[2 * b]
        elig = (bits[b] > t[b]) | ((bits[b] == t[b]) & (orig <= istar))
        score0.append(jnp.where(elig, prob[b], -1.0))

    lane = jax.lax.broadcasted_iota(jnp.int32, (1, 128), 1)

    def nms_body(i, score):
        new_score = []
        for b in range(nb):
            s = score[b]
            m = jnp.max(s)
            valid = m >= 0.0
            sel = s == m
            idxsel = jnp.min(jnp.where(sel, orig, jnp.int32(0x40000000)))
            onehot = sel & (orig == idxsel)
            bx1 = jnp.sum(jnp.where(onehot, x1[b], 0.0))
            by1 = jnp.sum(jnp.where(onehot, y1[b], 0.0))
            bx2 = jnp.sum(jnp.where(onehot, x2[b], 0.0))
            by2 = jnp.sum(jnp.where(onehot, y2[b], 0.0))
            ba = jnp.maximum(bx2 - bx1, 0.0) * jnp.maximum(by2 - by1, 0.0)
            xx1 = jnp.maximum(bx1, x1[b])
            yy1 = jnp.maximum(by1, y1[b])
            xx2 = jnp.minimum(bx2, x2[b])
            yy2 = jnp.minimum(by2, y2[b])
            inter = jnp.maximum(xx2 - xx1, 0.0) * jnp.maximum(yy2 - yy1, 0.0)
            iou = inter / jnp.maximum(ba + area[b] - inter, 1e-8)
            new_score.append(jnp.where((iou > _TH) | onehot, -1.0, s))
            v0 = jnp.where(valid, bx1, 0.0)
            v1 = jnp.where(valid, by1, 0.0)
            v2 = jnp.where(valid, bx2, 1.0)
            v3 = jnp.where(valid, by2, 1.0)
            row = jnp.where(lane == 0, v0,
                            jnp.where(lane == 1, v1,
                                      jnp.where(lane == 2, v2,
                                                jnp.where(lane == 3, v3, 0.0))))
            out_ref[b, pl.ds(i, 1), :] = row
        return tuple(new_score)

    jax.lax.fori_loop(0, _N_OUT, nms_body, tuple(score0))


def _anchor_consts():
    ratios = _RATIOS.astype(np.float32)
    ws = (np.float32(_SIZE) / np.sqrt(ratios)).astype(np.float32)
    hs = (np.float32(_SIZE) * np.sqrt(ratios)).astype(np.float32)
    sx = (np.arange(_W, dtype=np.float32) + np.float32(0.5)) * np.float32(_STRIDE)
    sy = (np.arange(_H, dtype=np.float32) + np.float32(0.5)) * np.float32(_STRIDE)
    cy, cx = np.meshgrid(sy, sx, indexing='ij')
    cx = cx[:, :, None].astype(np.float32)
    cy = cy[:, :, None].astype(np.float32)
    x1 = cx - ws[None, None, :] * np.float32(0.5)
    y1 = cy - hs[None, None, :] * np.float32(0.5)
    x2 = cx + ws[None, None, :] * np.float32(0.5)
    y2 = cy + hs[None, None, :] * np.float32(0.5)
    anch = np.stack([x1, y1, x2, y2], axis=-1).astype(np.float32).reshape(-1, 4)
    mine = anch.reshape(_H * _W, _A, 4).transpose(1, 0, 2).reshape(_N, 4)
    w = mine[:, 2] - mine[:, 0]
    h = mine[:, 3] - mine[:, 1]
    acx = mine[:, 0] + np.float32(0.5) * w
    acy = mine[:, 1] + np.float32(0.5) * h
    k = np.arange(_N, dtype=np.int32)
    orig = (k % (_H * _W)) * _A + k // (_H * _W)
    r = lambda v: v.reshape(_ROWS, 128)
    return (r(w.astype(np.float32)), r(h.astype(np.float32)),
            r(acx.astype(np.float32)), r(acy.astype(np.float32)),
            r(orig.astype(np.int32)))


_AW, _AH, _ACX, _ACY, _ORIG = _anchor_consts()


def kernel(features, img_info, gt_bboxes, W1, b1, Wc, bc, Wr, br):
    B = features.shape[0]
    feat = jnp.transpose(features, (0, 2, 3, 1)).reshape(B, _H * _W, _C_IN)
    col = jnp.arange(_H * _W) % _W
    fl = jnp.where((col == _W - 1)[None, :, None], 0.0, feat)
    fr = jnp.where((col == 0)[None, :, None], 0.0, feat)
    fl = jnp.pad(fl, ((0, 0), (_PAD + 1, _PAD - 1), (0, 0)))
    fc = jnp.pad(feat, ((0, 0), (_PAD, _PAD), (0, 0)))
    fr = jnp.pad(fr, ((0, 0), (_PAD - 1, _PAD + 1), (0, 0)))
    w1s = jnp.stack([W1[:, :, dy, dx].T for dy in range(3) for dx in range(3)])
    wh = jnp.concatenate([Wc.reshape(_A, _C_MID), Wr.reshape(4 * _A, _C_MID)], 0).T
    wh = jnp.pad(wh, ((0, 0), (0, 128 - 5 * _A)))
    bh = jnp.pad(jnp.concatenate([bc, br]), (0, 128 - 5 * _A)).reshape(1, 128)
    b1r = b1.reshape(1, _C_MID)

    npad = _H * _W + 2 * _PAD
    heads = pl.pallas_call(
        _trunk_body,
        grid=(B, _H * _W // _MBLK),
        in_specs=[
            pl.BlockSpec((1, npad, _C_IN), lambda b, m: (b, 0, 0)),
            pl.BlockSpec((1, npad, _C_IN), lambda b, m: (b, 0, 0)),
            pl.BlockSpec((1, npad, _C_IN), lambda b, m: (b, 0, 0)),
            pl.BlockSpec((9, _C_IN, _C_MID), lambda b, m: (0, 0, 0)),
            pl.BlockSpec((1, _C_MID), lambda b, m: (0, 0)),
            pl.BlockSpec((_C_MID, 128), lambda b, m: (0, 0)),
            pl.BlockSpec((1, 128), lambda b, m: (0, 0)),
        ],
        out_specs=pl.BlockSpec((1, _MBLK, 128), lambda b, m: (b, m, 0)),
        out_shape=jax.ShapeDtypeStruct((B, _H * _W, 128), jnp.float32),
    )(fl, fc, fr, w1s, b1r, wh, bh)

    tr = lambda s: heads[:, :, s].transpose(0, 2, 1).reshape(B, _ROWS, 128)
    cls_my = tr(slice(0, 3))
    rcx = tr(slice(3, 15, 4))
    rcy = tr(slice(4, 15, 4))
    rw = tr(slice(5, 15, 4))
    rh = tr(slice(6, 15, 4))

    cspec = pl.BlockSpec((_ROWS, 128), lambda: (0, 0))
    bspec = pl.BlockSpec((B, _ROWS, 128), lambda: (0, 0, 0))
    out = pl.pallas_call(
        _nms_body,
        in_specs=[
            pl.BlockSpec((B, 1, 3), lambda: (0, 0, 0), memory_space=pltpu.SMEM),
            bspec, bspec, bspec, bspec, bspec,
            cspec, cspec, cspec, cspec, cspec,
        ],
        out_specs=pl.BlockSpec((B, 304, 128), lambda: (0, 0, 0)),
        out_shape=jax.ShapeDtypeStruct((B, 304, 128), jnp.float32),
    )(img_info.reshape(B, 1, 3), cls_my, rcx, rcy, rw, rh,
      jnp.asarray(_AW), jnp.asarray(_AH), jnp.asarray(_ACX), jnp.asarray(_ACY),
      jnp.asarray(_ORIG))
    return out[:, :_N_OUT, :4]


# speedup vs baseline: 7.7390x; 1.1429x over previous
import numpy as np
import jax
import jax.numpy as jnp
from jax.experimental import pallas as pl
from jax.experimental.pallas import tpu as pltpu

_RATIOS = np.array([0.5, 1.0, 2.0], np.float32)
_STRIDE = 8
_SIZE = 64
_K_PRE = 6000
_N_OUT = 300
_TH = 0.7
_H = 64
_W = 64
_A = 3
_C_IN = 256
_C_MID = 512
_N = _H * _W * _A
_ROWS = _N // 128
_MBLK = 1024
_PAD = 72


def _trunk_body(fl_ref, fc_ref, fr_ref, w1_ref, b1_ref, wh_ref, bh_ref, out_ref):
    mb = pl.program_id(1)
    base = pl.multiple_of(_PAD + mb * _MBLK, 8)
    variants = (fl_ref, fc_ref, fr_ref)
    acc = jnp.zeros((_MBLK, _C_MID), jnp.float32)
    for dy in range(3):
        for dx in range(3):
            src = variants[dx]
            start = pl.multiple_of(base + (dy - 1) * _W, 8)
            a = src[0, pl.ds(start, _MBLK), :]
            acc = acc + jnp.dot(a, w1_ref[dy * 3 + dx],
                                preferred_element_type=jnp.float32)
    x = jnp.maximum(acc + b1_ref[0:1, :], 0.0)
    heads = jnp.dot(x, wh_ref[...], preferred_element_type=jnp.float32)
    out_ref[0] = heads + bh_ref[0:1, :]


_NCH = _ROWS // 8


def _pairtree(items, comb):
    cur = list(items)
    while len(cur) > 1:
        nxt = [comb(cur[i], cur[i + 1]) for i in range(0, len(cur) - 1, 2)]
        if len(cur) % 2:
            nxt.append(cur[-1])
        cur = nxt
    return cur[0]


def _sumtree(chunks):
    t = _pairtree(chunks, lambda a, b: a + b)
    for sh in (4, 2, 1):
        t = t + jnp.roll(t, sh, axis=0)
    for sh in (64, 32, 16, 8, 4, 2, 1):
        t = t + jnp.roll(t, sh, axis=1)
    return t


def _comb(A, B):
    ta = (A[0] > B[0]) | ((A[0] == B[0]) & (A[1] < B[1]))
    return tuple(jnp.where(ta, x, y) for x, y in zip(A, B))


def _tournament(tuples):
    T = _pairtree(tuples, _comb)
    for sh in (4, 2, 1):
        T = _comb(T, tuple(jnp.roll(x, sh, axis=0) for x in T))
    for sh in (64, 32, 16, 8, 4, 2, 1):
        T = _comb(T, tuple(jnp.roll(x, sh, axis=1) for x in T))
    return T


def _nms_body(info_ref, cls_ref, rcx_ref, rcy_ref, rw_ref, rh_ref,
              aw_ref, ah_ref, acx_ref, acy_ref, orig_ref, out_ref):
    nb = cls_ref.shape[0]
    awc, ahc, acxc, acyc, origc = [], [], [], [], []
    for r in range(_NCH):
        awc.append(aw_ref[r * 8:(r + 1) * 8, :])
        ahc.append(ah_ref[r * 8:(r + 1) * 8, :])
        acxc.append(acx_ref[r * 8:(r + 1) * 8, :])
        acyc.append(acy_ref[r * 8:(r + 1) * 8, :])
        origc.append(orig_ref[r * 8:(r + 1) * 8, :])

    prob, x1, y1, x2, y2, area, bits = [[] for _ in range(7)]
    for b in range(nb):
        ih = info_ref[b, 0, 0]
        iw = info_ref[b, 0, 1]
        pb, x1b, y1b, x2b, y2b, arb, btb = [[] for _ in range(7)]
        for r in range(_NCH):
            sl = slice(r * 8, (r + 1) * 8)
            p = jax.nn.sigmoid(cls_ref[b, sl, :])
            pcx = rcx_ref[b, sl, :] * awc[r] + acxc[r]
            pcy = rcy_ref[b, sl, :] * ahc[r] + acyc[r]
            pw = jnp.exp(jnp.clip(rw_ref[b, sl, :], -10.0, 10.0)) * awc[r]
            ph = jnp.exp(jnp.clip(rh_ref[b, sl, :], -10.0, 10.0)) * ahc[r]
            bx1 = jnp.clip(pcx - 0.5 * pw, 0.0, iw)
            by1 = jnp.clip(pcy - 0.5 * ph, 0.0, ih)
            bx2 = jnp.clip(pcx + 0.5 * pw, 0.0, iw)
            by2 = jnp.clip(pcy + 0.5 * ph, 0.0, ih)
            pb.append(p)
            x1b.append(bx1)
            y1b.append(by1)
            x2b.append(bx2)
            y2b.append(by2)
            arb.append(jnp.maximum(bx2 - bx1, 0.0) * jnp.maximum(by2 - by1, 0.0))
            btb.append(jax.lax.bitcast_convert_type(p, jnp.int32))
        prob.append(pb)
        x1.append(x1b)
        y1.append(y1b)
        x2.append(x2b)
        y2.append(y2b)
        area.append(arb)
        bits.append(btb)

    def bs_body(_, c):
        out = []
        for b in range(nb):
            lo, hi = c[2 * b], c[2 * b + 1]
            mid = (lo + hi + 1) >> 1
            cnt = _sumtree([(bits[b][r] >= mid).astype(jnp.int32)
                            for r in range(_NCH)])
            big = cnt >= _K_PRE
            out += [jnp.where(big, mid, lo), jnp.where(big, hi, mid - 1)]
        return tuple(out)

    c0 = (jnp.zeros((8, 128), jnp.int32),
          jnp.full((8, 128), 0x3F800000, jnp.int32)) * nb
    cr = jax.lax.fori_loop(0, 31, bs_body, c0)
    t = [cr[2 * b] for b in range(nb)]
    quota = [_K_PRE - _sumtree([(bits[b][r] > t[b]).astype(jnp.int32)
                                for r in range(_NCH)]) for b in range(nb)]

    def bs2_body(_, c):
        out = []
        for b in range(nb):
            lo, hi = c[2 * b], c[2 * b + 1]
            mid = (lo + hi) >> 1
            cnt = _sumtree([((bits[b][r] == t[b]) & (origc[r] <= mid)).astype(jnp.int32)
                            for r in range(_NCH)])
            ok = cnt >= quota[b]
            out += [jnp.where(ok, lo, mid + 1), jnp.where(ok, mid, hi)]
        return tuple(out)

    c0 = (jnp.full((8, 128), -1, jnp.int32),
          jnp.full((8, 128), _N - 1, jnp.int32)) * nb
    cr = jax.lax.fori_loop(0, 14, bs2_body, c0)
    score0 = []
    for b in range(nb):
        istar = cr[2 * b]
        for r in range(_NCH):
            elig = (bits[b][r] > t[b]) | ((bits[b][r] == t[b]) & (origc[r] <= istar))
            score0.append(jnp.where(elig, prob[b][r], -1.0))

    lane = jax.lax.broadcasted_iota(jnp.int32, (1, 128), 1)
    padrow = jnp.where((lane == 2) | (lane == 3), 1.0, 0.0)

    def nms_body(i, score):
        new_score = []
        for b in range(nb):
            s = score[b * _NCH:(b + 1) * _NCH]
            ms, mo, bx1, by1, bx2, by2 = _tournament(
                [(s[r], origc[r], x1[b][r], y1[b][r], x2[b][r], y2[b][r])
                 for r in range(_NCH)])
            ba = jnp.maximum(bx2 - bx1, 0.0) * jnp.maximum(by2 - by1, 0.0)
            for r in range(_NCH):
                xx1 = jnp.maximum(bx1, x1[b][r])
                yy1 = jnp.maximum(by1, y1[b][r])
                xx2 = jnp.minimum(bx2, x2[b][r])
                yy2 = jnp.minimum(by2, y2[b][r])
                inter = jnp.maximum(xx2 - xx1, 0.0) * jnp.maximum(yy2 - yy1, 0.0)
                iou = inter / jnp.maximum(ba + area[b][r] - inter, 1e-8)
                new_score.append(jnp.where((iou > _TH) | (origc[r] == mo),
                                           -1.0, s[r]))
            valid = ms[0:1] >= 0.0
            row = jnp.where(lane == 0, bx1[0:1],
                            jnp.where(lane == 1, by1[0:1],
                                      jnp.where(lane == 2, bx2[0:1],
                                                jnp.where(lane == 3, by2[0:1],
                                                          0.0))))
            out_ref[b, pl.ds(i, 1), :] = jnp.where(valid, row, padrow)
        return tuple(new_score)

    jax.lax.fori_loop(0, _N_OUT, nms_body, tuple(score0))


def _anchor_consts():
    ratios = _RATIOS.astype(np.float32)
    ws = (np.float32(_SIZE) / np.sqrt(ratios)).astype(np.float32)
    hs = (np.float32(_SIZE) * np.sqrt(ratios)).astype(np.float32)
    sx = (np.arange(_W, dtype=np.float32) + np.float32(0.5)) * np.float32(_STRIDE)
    sy = (np.arange(_H, dtype=np.float32) + np.float32(0.5)) * np.float32(_STRIDE)
    cy, cx = np.meshgrid(sy, sx, indexing='ij')
    cx = cx[:, :, None].astype(np.float32)
    cy = cy[:, :, None].astype(np.float32)
    x1 = cx - ws[None, None, :] * np.float32(0.5)
    y1 = cy - hs[None, None, :] * np.float32(0.5)
    x2 = cx + ws[None, None, :] * np.float32(0.5)
    y2 = cy + hs[None, None, :] * np.float32(0.5)
    anch = np.stack([x1, y1, x2, y2], axis=-1).astype(np.float32).reshape(-1, 4)
    mine = anch.reshape(_H * _W, _A, 4).transpose(1, 0, 2).reshape(_N, 4)
    w = mine[:, 2] - mine[:, 0]
    h = mine[:, 3] - mine[:, 1]
    acx = mine[:, 0] + np.float32(0.5) * w
    acy = mine[:, 1] + np.float32(0.5) * h
    k = np.arange(_N, dtype=np.int32)
    orig = (k % (_H * _W)) * _A + k // (_H * _W)
    r = lambda v: v.reshape(_ROWS, 128)
    return (r(w.astype(np.float32)), r(h.astype(np.float32)),
            r(acx.astype(np.float32)), r(acy.astype(np.float32)),
            r(orig.astype(np.int32)))


_AW, _AH, _ACX, _ACY, _ORIG = _anchor_consts()


def kernel(features, img_info, gt_bboxes, W1, b1, Wc, bc, Wr, br):
    B = features.shape[0]
    feat = jnp.transpose(features, (0, 2, 3, 1)).reshape(B, _H * _W, _C_IN)
    col = jnp.arange(_H * _W) % _W
    fl = jnp.where((col == _W - 1)[None, :, None], 0.0, feat)
    fr = jnp.where((col == 0)[None, :, None], 0.0, feat)
    fl = jnp.pad(fl, ((0, 0), (_PAD + 1, _PAD - 1), (0, 0)))
    fc = jnp.pad(feat, ((0, 0), (_PAD, _PAD), (0, 0)))
    fr = jnp.pad(fr, ((0, 0), (_PAD - 1, _PAD + 1), (0, 0)))
    w1s = jnp.stack([W1[:, :, dy, dx].T for dy in range(3) for dx in range(3)])
    wh = jnp.concatenate([Wc.reshape(_A, _C_MID), Wr.reshape(4 * _A, _C_MID)], 0).T
    wh = jnp.pad(wh, ((0, 0), (0, 128 - 5 * _A)))
    bh = jnp.pad(jnp.concatenate([bc, br]), (0, 128 - 5 * _A)).reshape(1, 128)
    b1r = b1.reshape(1, _C_MID)

    npad = _H * _W + 2 * _PAD
    heads = pl.pallas_call(
        _trunk_body,
        grid=(B, _H * _W // _MBLK),
        in_specs=[
            pl.BlockSpec((1, npad, _C_IN), lambda b, m: (b, 0, 0)),
            pl.BlockSpec((1, npad, _C_IN), lambda b, m: (b, 0, 0)),
            pl.BlockSpec((1, npad, _C_IN), lambda b, m: (b, 0, 0)),
            pl.BlockSpec((9, _C_IN, _C_MID), lambda b, m: (0, 0, 0)),
            pl.BlockSpec((1, _C_MID), lambda b, m: (0, 0)),
            pl.BlockSpec((_C_MID, 128), lambda b, m: (0, 0)),
            pl.BlockSpec((1, 128), lambda b, m: (0, 0)),
        ],
        out_specs=pl.BlockSpec((1, _MBLK, 128), lambda b, m: (b, m, 0)),
        out_shape=jax.ShapeDtypeStruct((B, _H * _W, 128), jnp.float32),
    )(fl, fc, fr, w1s, b1r, wh, bh)

    tr = lambda s: heads[:, :, s].transpose(0, 2, 1).reshape(B, _ROWS, 128)
    cls_my = tr(slice(0, 3))
    rcx = tr(slice(3, 15, 4))
    rcy = tr(slice(4, 15, 4))
    rw = tr(slice(5, 15, 4))
    rh = tr(slice(6, 15, 4))

    cspec = pl.BlockSpec((_ROWS, 128), lambda: (0, 0))
    bspec = pl.BlockSpec((B, _ROWS, 128), lambda: (0, 0, 0))
    out = pl.pallas_call(
        _nms_body,
        in_specs=[
            pl.BlockSpec((B, 1, 3), lambda: (0, 0, 0), memory_space=pltpu.SMEM),
            bspec, bspec, bspec, bspec, bspec,
            cspec, cspec, cspec, cspec, cspec,
        ],
        out_specs=pl.BlockSpec((B, 304, 128), lambda: (0, 0, 0)),
        out_shape=jax.ShapeDtypeStruct((B, 304, 128), jnp.float32),
    )(img_info.reshape(B, 1, 3), cls_my, rcx, rcy, rw, rh,
      jnp.asarray(_AW), jnp.asarray(_AH), jnp.asarray(_ACX), jnp.asarray(_ACY),
      jnp.asarray(_ORIG))
    return out[:, :_N_OUT, :4]
